# baseline (speedup 1.0000x reference)
import functools

import jax
import jax.numpy as jnp
from jax import lax
from jax.experimental import pallas as pl
from jax.experimental.pallas import tpu as pltpu
from jax.experimental.pallas import tpu_sc as plsc

_EPS = 1e-3


def _sc_gather(table, idxs):
    _, emb = table.shape
    batch = idxs.shape[0]
    info = plsc.get_sparse_core_info()
    num_workers = info.num_cores * info.num_subcores
    bpw = batch // num_workers
    mesh = plsc.VectorSubcoreMesh(core_axis_name="c", subcore_axis_name="s")

    @functools.partial(
        pl.kernel,
        mesh=mesh,
        out_type=jax.ShapeDtypeStruct((batch, emb), jnp.float32),
        scratch_types=[
            pltpu.VMEM((bpw,), jnp.int32),
            pltpu.VMEM((bpw, emb), jnp.float32),
            pltpu.SemaphoreType.DMA,
        ],
    )
    def gather_k(table_hbm, idx_hbm, out_hbm, idx_v, rows_v, sem):
        wid = lax.axis_index("s") * info.num_cores + lax.axis_index("c")
        base = wid * bpw
        pltpu.sync_copy(idx_hbm.at[pl.ds(base, bpw)], idx_v)
        pltpu.async_copy(table_hbm.at[idx_v], rows_v, sem).wait()
        pltpu.sync_copy(rows_v, out_hbm.at[pl.ds(base, bpw)])

    return gather_k(table, idxs)


def _fused_body(w_ref, x_ref, cents_ref, out_ref,
                *, n, nchunks, bn, nd, sub, kk):
    kt = kk // 4
    f32 = jnp.float32

    n2s = []
    cents2 = []
    for d in range(nd):
        cd = cents_ref[d]
        n2s.append(jnp.sum(cd * cd, axis=1)[None, :])
        cents2.append(cd + cd)
    ones_s = jnp.ones((sub, 1), f32)

    def resp(base, d, sup):
        z = x_ref[pl.ds(base, bn), d * sub:(d + 1) * sub]
        n1 = lax.dot_general(z * z, ones_s, (((1,), (0,)), ((), ())),
                             precision=lax.Precision.HIGHEST)
        dt2 = lax.dot_general(z, cents2[d], (((1,), (1,)), ((), ())),
                              precision=lax.Precision.DEFAULT)
        return (dt2 - n1) - sup, z

    tot = resp(0, 0, n2s[0])[0]
    for d in range(1, nd):
        tot = tot + resp(0, d, n2s[d])[0]
    c = jnp.sum(tot, axis=0, keepdims=True) / (float(nd) * bn)
    n2cs = [n2s[d] + c for d in range(nd)]

    def p1(i, carry):
        s1, s2, s1h, s2h, cnth = carry
        base = i * bn
        w = w_ref[pl.ds(base, bn), :].astype(f32)
        for d in range(nd):
            rc = resp(base, d, n2cs[d])[0]
            rc2 = rc * rc
            s1 = s1 + jnp.sum(rc, axis=0, keepdims=True)
            s2 = s2 + jnp.sum(rc2, axis=0, keepdims=True)
            s1h = s1h + jnp.sum(rc * w, axis=0, keepdims=True)
            s2h = s2h + jnp.sum(rc2 * w, axis=0, keepdims=True)
        cnth = cnth + jnp.sum(w)
        return (s1, s2, s1h, s2h, cnth)

    zrow = jnp.zeros((1, kk), f32)
    s1, s2, s1h, s2h, cnth = lax.fori_loop(
        0, nchunks, p1, (zrow, zrow, zrow, zrow, f32(0.0)))

    denh = cnth * float(nd)
    dent = (float(n) - cnth) * float(nd)
    mh_c = s1h / denh
    sh = jnp.sqrt(s2h / denh - mh_c * mh_c + _EPS)
    mh = c + mh_c
    invh = 1.0 / sh
    mt_c = (s1 - s1h) / dent
    stt = jnp.sqrt((s2 - s2h) / dent - mt_c * mt_c + _EPS)
    mt = (c + mt_c)[:, :kt]
    invt = (1.0 / stt)[:, :kt]

    chi = [cents_ref[d].astype(jnp.bfloat16) for d in range(nd)]
    clo = [(cents_ref[d] - chi[d].astype(f32)).astype(jnp.bfloat16)
           for d in range(nd)]

    zpad = jnp.zeros((bn, kk - kt), jnp.bfloat16)

    def p2(i, _):
        base = i * bn
        head = w_ref[pl.ds(base, bn), :] > 0
        for d in range(nd):
            r, z = resp(base, d, n2s[d])
            rh = (r - mh) * invh
            ohh = (rh == jnp.max(rh, axis=1, keepdims=True)
                   ).astype(jnp.bfloat16)
            rt = (r[:, :kt] - mt) * invt
            oht = (rt == jnp.max(rt, axis=1, keepdims=True)
                   ).astype(jnp.bfloat16)
            ohtp = jnp.concatenate([oht, zpad], axis=1)
            oh = jnp.where(head, ohh, ohtp)
            od = (lax.dot_general(oh, chi[d], (((1,), (0,)), ((), ())),
                                  preferred_element_type=f32)
                  + lax.dot_general(oh, clo[d], (((1,), (0,)), ((), ())),
                                    preferred_element_type=f32))
            out_ref[pl.ds(base, bn), d * sub:(d + 1) * sub] = (od - z) + z
        return 0

    lax.fori_loop(0, nchunks, p2, 0)


def kernel(table, centroids, indices):
    vocab, emb = table.shape
    nd, kk, sub = centroids.shape
    cutoff = int(vocab * 0.8)
    idxs = indices.reshape(-1)
    n = idxs.shape[0]

    x = _sc_gather(table, idxs)
    wcol = (idxs >= cutoff).astype(jnp.bfloat16)[:, None]

    bn = 2048
    nchunks = n // bn
    out = pl.pallas_call(
        functools.partial(_fused_body, n=n, nchunks=nchunks, bn=bn,
                          nd=nd, sub=sub, kk=kk),
        out_shape=jax.ShapeDtypeStruct((n, emb), jnp.float32),
    )(wcol, x, centroids)

    return out.reshape(indices.shape + (emb,))
